# baseline (speedup 1.0000x reference)
import numpy as np
import jax
import jax.numpy as jnp
from jax import lax
from jax.experimental import pallas as pl
from jax.experimental.pallas import tpu as pltpu
from jax.experimental.pallas import tpu_sc as plsc

_M, _D, _B = 1048576, 32, 16384
_NC, _NS = 2, 16
_CB = _B // (_NC * _NS)

_JOTA = np.arange(_B, dtype=np.int32)


def _vm_body(idx_hbm, jota_hbm, val_hbm, out_hbm,
             tbl_s, idx_all, j_all, my_idx, w_idx, rows,
             sem, sem_a, sem_b):
    c = lax.axis_index("c")
    s = lax.axis_index("s")
    base = (c * _NS + s) * _CB

    cp_idx = pltpu.async_copy(idx_hbm.at[pl.ds(base, _CB)], my_idx, sem)

    @pl.when(s == 0)
    def _():
        cp_a = pltpu.async_copy(idx_hbm, idx_all, sem_a)
        cp_b = pltpu.async_copy(jota_hbm, j_all, sem_b)
        cp_a.wait()
        cp_b.wait()
        pltpu.sync_copy(j_all, tbl_s.at[idx_all])

    cp_idx.wait()
    plsc.subcore_barrier()

    pltpu.sync_copy(tbl_s.at[my_idx], w_idx)
    pltpu.async_copy(val_hbm.at[w_idx], rows, sem).wait()
    pltpu.sync_copy(rows, out_hbm.at[pl.ds(base, _CB)])


@jax.jit
def _vm_call(idx, val):
    mesh = plsc.VectorSubcoreMesh(core_axis_name="c", subcore_axis_name="s")
    return pl.kernel(
        _vm_body,
        out_type=jax.ShapeDtypeStruct((_B, _D), jnp.float32),
        mesh=mesh,
        compiler_params=pltpu.CompilerParams(use_tc_tiling_on_sc=False),
        scratch_types=[
            pltpu.VMEM_SHARED((_M,), jnp.int32),
            pltpu.VMEM((_B,), jnp.int32),
            pltpu.VMEM((_B,), jnp.int32),
            pltpu.VMEM((_CB,), jnp.int32),
            pltpu.VMEM((_CB,), jnp.int32),
            pltpu.VMEM((_CB, _D), jnp.float32),
            pltpu.SemaphoreType.DMA,
            pltpu.SemaphoreType.DMA,
            pltpu.SemaphoreType.DMA,
        ],
    )(idx, jnp.asarray(_JOTA), val)


def kernel(mem, idx, val):
    del mem
    return _vm_call(idx, val)


# speedup vs baseline: 1.0002x; 1.0002x over previous
import numpy as np
import jax
import jax.numpy as jnp
from jax import lax
from jax.experimental import pallas as pl
from jax.experimental.pallas import tpu as pltpu
from jax.experimental.pallas import tpu_sc as plsc

_M, _D, _B = 1048576, 32, 16384
_NC, _NS = 2, 16
_CB = _B // (_NC * _NS)

_JOTA = np.arange(_B, dtype=np.int32)


def _vm_body(idx_hbm, jota_hbm, val_hbm, out_hbm,
             tbl_s, idx_all, j_all, my_idx, w_idx, rows, sem, sem_a, sem_b):
    c = lax.axis_index("c")
    s = lax.axis_index("s")
    base = (c * _NS + s) * _CB

    cp_idx = pltpu.async_copy(idx_hbm.at[pl.ds(base, _CB)], my_idx, sem)

    @pl.when(s == 0)
    def _():
        cp_a = pltpu.async_copy(idx_hbm, idx_all, sem_a)
        cp_b = pltpu.async_copy(jota_hbm, j_all, sem_b)
        cp_a.wait()
        cp_b.wait()
        pltpu.sync_copy(j_all, tbl_s.at[idx_all])

    cp_idx.wait()
    plsc.subcore_barrier()

    pltpu.sync_copy(tbl_s.at[my_idx], w_idx)
    pltpu.async_copy(val_hbm.at[w_idx], rows, sem).wait()
    pltpu.sync_copy(rows, out_hbm.at[pl.ds(base, _CB)])


@jax.jit
def _vm_call(idx, jota, val):
    mesh = plsc.VectorSubcoreMesh(core_axis_name="c", subcore_axis_name="s")
    return pl.kernel(
        _vm_body,
        out_type=jax.ShapeDtypeStruct((_B, _D), jnp.float32),
        mesh=mesh,
        compiler_params=pltpu.CompilerParams(use_tc_tiling_on_sc=False),
        scratch_types=[
            pltpu.VMEM_SHARED((_M,), jnp.int32),
            pltpu.VMEM((_B,), jnp.int32),
            pltpu.VMEM((_B,), jnp.int32),
            pltpu.VMEM((_CB,), jnp.int32),
            pltpu.VMEM((_CB,), jnp.int32),
            pltpu.VMEM((_CB, _D), jnp.float32),
            pltpu.SemaphoreType.DMA,
            pltpu.SemaphoreType.DMA,
            pltpu.SemaphoreType.DMA,
        ],
    )(idx, jota, val)


def kernel(mem, idx, val):
    del mem
    return _vm_call(idx, jnp.asarray(_JOTA), val)
